# baseline (speedup 1.0000x reference)
import functools

import jax
import jax.numpy as jnp
from jax import lax
from jax.experimental import pallas as pl
from jax.experimental.pallas import tpu as pltpu
from jax.experimental.pallas import tpu_sc as plsc

_B, _T, _D = 4096, 125, 128
_TP = 128
_TFULL = 120
_NC, _NS = 2, 16
_NW = _NC * _NS
_BPW = _B // _NW
_LANES = 16
_DV = _D // _LANES
_NBUF = 4

_mesh = plsc.VectorSubcoreMesh(
    core_axis_name="c", subcore_axis_name="s", num_cores=_NC, num_subcores=_NS
)


@functools.partial(
    pl.kernel,
    out_type=(
        jax.ShapeDtypeStruct((_B, _T, _D), jnp.float32),
        jax.ShapeDtypeStruct((_B, _D), jnp.float32),
        jax.ShapeDtypeStruct((_B, _D), jnp.float32),
    ),
    mesh=_mesh,
    compiler_params=pltpu.CompilerParams(use_tc_tiling_on_sc=True),
    scratch_types=[
        pltpu.VMEM((_BPW, _TP), jnp.int32),
        pltpu.VMEM((_TP, _D), jnp.float32),
        pltpu.VMEM((8, _D), jnp.float32),
        [pltpu.VMEM((_TP, _D), jnp.float32)] * _NBUF,
        pltpu.VMEM((_BPW, _D), jnp.float32),
        [pltpu.SemaphoreType.DMA] * _NBUF,
        [pltpu.SemaphoreType.DMA] * _NBUF,
    ],
)
def _embed_sc(zeo, syn, idx_hbm, table, te_hbm, pe_hbm,
              out_seq, out_zeo, out_syn,
              idx_v, bias_v, te_v, rows, zs_v, gsem, osem):
    wid = lax.axis_index("s") * _NC + lax.axis_index("c")
    base = wid * _BPW

    pltpu.sync_copy(te_hbm, te_v)
    pltpu.sync_copy(pe_hbm, bias_v)
    pltpu.sync_copy(idx_hbm.at[pl.ds(base, _BPW)], idx_v)

    def bias_body(t8, carry):
        for u in range(8):
            t = t8 * 8 + u
            for d in range(_DV):
                sl = pl.ds(d * _LANES, _LANES)
                plsc.addupdate(bias_v.at[t, sl], te_v[2, sl])
        return carry
    lax.fori_loop(0, _TP // 8, bias_body, 0)

    for src, dst, row in ((zeo, out_zeo, 0), (syn, out_syn, 1)):
        pltpu.sync_copy(src.at[pl.ds(base, _BPW)], zs_v)
        def zs_body(i, carry, row=row):
            for d in range(_DV):
                sl = pl.ds(d * _LANES, _LANES)
                plsc.addupdate(zs_v.at[i, sl], te_v[row, sl])
            return carry
        lax.fori_loop(0, _BPW, zs_body, 0)
        pltpu.sync_copy(zs_v, dst.at[pl.ds(base, _BPW)])

    def g_copy(k, j):
        return pltpu.make_async_copy(table.at[idx_v.at[k]], rows[j], gsem[j])

    def o_copies(k, j):
        return (
            pltpu.make_async_copy(
                rows[j].at[pl.ds(0, _TFULL)],
                out_seq.at[base + k, pl.ds(0, _TFULL)], osem[j]),
            pltpu.make_async_copy(
                rows[j].at[pl.ds(_TFULL, _T - _TFULL)],
                out_seq.at[base + k, pl.ds(_TFULL, _T - _TFULL)], osem[j]),
        )

    def o_start(k, j):
        for c in o_copies(k, j):
            c.start()

    def o_wait(k, j):
        for c in o_copies(k, j):
            c.wait()

    def add_bias(k, j):
        def add_body(t8, carry):
            for u in range(8):
                t = t8 * 8 + u
                for d in range(_DV):
                    sl = pl.ds(d * _LANES, _LANES)
                    plsc.addupdate(rows[j].at[t, sl], bias_v[t, sl])
            return carry
        lax.fori_loop(0, _TP // 8, add_body, 0)

    g_copy(0, 0).start()
    g_copy(1, 1).start()
    for k in (0, 1):
        g_copy(k, k).wait()
        add_bias(k, k)
        o_start(k, k)
        g_copy(k + 2, k + 2).start()

    def main_body(k4, carry):
        for j in range(_NBUF):
            k = 2 + k4 * _NBUF + j
            buf = (2 + j) % _NBUF
            nbuf = j % _NBUF
            g_copy(k, buf).wait()
            add_bias(k, buf)
            o_start(k, buf)
            o_wait(k - 2, nbuf)
            g_copy(k + 2, nbuf).start()
        return carry
    lax.fori_loop(0, (_BPW - _NBUF) // _NBUF, main_body, 0)

    for k in (_BPW - 2, _BPW - 1):
        j = k % _NBUF
        g_copy(k, j).wait()
        add_bias(k, j)
        o_start(k, j)
    for k in range(_BPW - _NBUF, _BPW):
        o_wait(k, k % _NBUF)


def kernel(zeo, syn, smis_seq, char_embed, type_embed, pe):
    idx = jnp.pad(smis_seq.astype(jnp.int32), ((0, 0), (0, _TP - _T)))
    pe_pad = jnp.pad(pe.reshape(_T, _D), ((0, _TP - _T), (0, 0)))
    te_pad = jnp.pad(type_embed, ((0, 8 - type_embed.shape[0]), (0, 0)))
    zeo2d = zeo.reshape(_B, _D)
    syn2d = syn.reshape(_B, _D)
    out_seq, out_zeo, out_syn = _embed_sc(
        zeo2d, syn2d, idx, char_embed, te_pad, pe_pad)
    return out_seq, out_zeo.reshape(_B, 1, _D), out_syn.reshape(_B, 1, _D)


# speedup vs baseline: 2.9607x; 2.9607x over previous
import functools

import jax
import jax.numpy as jnp
from jax import lax
from jax.experimental import pallas as pl
from jax.experimental.pallas import tpu as pltpu
from jax.experimental.pallas import tpu_sc as plsc

_B, _T, _D = 4096, 125, 128
_V = 1000
_NC, _NS = 2, 16
_NW = _NC * _NS
_BPW = _B // _NW
_LANES = 16
_DV = _D // _LANES
_NBUF = 4

_mesh = plsc.VectorSubcoreMesh(
    core_axis_name="c", subcore_axis_name="s", num_cores=_NC, num_subcores=_NS
)


@functools.partial(
    pl.kernel,
    out_type=(
        jax.ShapeDtypeStruct((_B, _T, _D), jnp.float32),
        jax.ShapeDtypeStruct((_B, 1, _D), jnp.float32),
        jax.ShapeDtypeStruct((_B, 1, _D), jnp.float32),
    ),
    mesh=_mesh,
    scratch_types=[
        pltpu.VMEM_SHARED((_V, _D), jnp.float32),
        pltpu.VMEM((_BPW, _T), jnp.int32),
        pltpu.VMEM((_T, _D), jnp.float32),
        pltpu.VMEM((3, _D), jnp.float32),
        [pltpu.VMEM((_T, _D), jnp.float32)] * _NBUF,
        pltpu.VMEM((_BPW, 1, _D), jnp.float32),
        [pltpu.SemaphoreType.DMA] * _NBUF,
        [pltpu.SemaphoreType.DMA] * _NBUF,
    ],
)
def _embed_sc(zeo, syn, idx_hbm, table, te_hbm, pe_hbm,
              out_seq, out_zeo, out_syn,
              table_sh, idx_v, bias_v, te_v, rows, zs_v, gsem, osem):
    sid = lax.axis_index("s")
    wid = sid * _NC + lax.axis_index("c")
    base = wid * _BPW

    @pl.when(sid == 0)
    def _():
        pltpu.sync_copy(table, table_sh)

    pltpu.sync_copy(te_hbm, te_v)
    pltpu.sync_copy(pe_hbm, bias_v)
    pltpu.sync_copy(idx_hbm.at[pl.ds(base, _BPW)], idx_v)

    def bias_body(t5, carry):
        for u in range(5):
            t = t5 * 5 + u
            for d in range(_DV):
                sl = pl.ds(d * _LANES, _LANES)
                plsc.addupdate(bias_v.at[t, sl], te_v[2, sl])
        return carry
    lax.fori_loop(0, _T // 5, bias_body, 0)

    for src, dst, row in ((zeo, out_zeo, 0), (syn, out_syn, 1)):
        pltpu.sync_copy(src.at[pl.ds(base, _BPW)], zs_v)
        def zs_body(i, carry, row=row):
            for d in range(_DV):
                sl = pl.ds(d * _LANES, _LANES)
                plsc.addupdate(zs_v.at[i, 0, sl], te_v[row, sl])
            return carry
        lax.fori_loop(0, _BPW, zs_body, 0)
        pltpu.sync_copy(zs_v, dst.at[pl.ds(base, _BPW)])

    plsc.subcore_barrier()

    def g_copy(k, j):
        return pltpu.make_async_copy(
            table_sh.at[idx_v.at[k]], rows[j], gsem[j])

    def o_copy(k, j):
        return pltpu.make_async_copy(rows[j], out_seq.at[base + k], osem[j])

    def add_bias(k, j):
        def add_body(t5, carry):
            for u in range(5):
                t = t5 * 5 + u
                for d in range(_DV):
                    sl = pl.ds(d * _LANES, _LANES)
                    plsc.addupdate(rows[j].at[t, sl], bias_v[t, sl])
            return carry
        lax.fori_loop(0, _T // 5, add_body, 0)

    g_copy(0, 0).start()
    g_copy(1, 1).start()
    for k in (0, 1):
        g_copy(k, k).wait()
        add_bias(k, k)
        o_copy(k, k).start()
        g_copy(k + 2, k + 2).start()

    def main_body(k4, carry):
        for j in range(_NBUF):
            k = 2 + k4 * _NBUF + j
            buf = (2 + j) % _NBUF
            nbuf = j % _NBUF
            g_copy(k, buf).wait()
            add_bias(k, buf)
            o_copy(k, buf).start()
            o_copy(k - 2, nbuf).wait()
            g_copy(k + 2, nbuf).start()
        return carry
    lax.fori_loop(0, (_BPW - _NBUF) // _NBUF, main_body, 0)

    for k in (_BPW - 2, _BPW - 1):
        j = k % _NBUF
        g_copy(k, j).wait()
        add_bias(k, j)
        o_copy(k, j).start()
    for k in range(_BPW - _NBUF, _BPW):
        o_copy(k, k % _NBUF).wait()


def kernel(zeo, syn, smis_seq, char_embed, type_embed, pe):
    idx = smis_seq.astype(jnp.int32)
    pe2d = pe.reshape(_T, _D)
    return _embed_sc(zeo, syn, idx, char_embed, type_embed, pe2d)
